# baseline (device time: 11112 ns/iter reference)
import jax
import jax.numpy as jnp
from jax import lax
from jax.experimental import pallas as pl
from jax.experimental.pallas import tpu as pltpu

N_DEV = 4
N_GLOBAL = 2048
EPS = 1e-5
N_CHUNK = 4


def kernel(x, gamma, beta):
    m, n_per = x.shape
    rows = m // N_CHUNK

    def body(x_hbm, gamma_hbm, beta_hbm, out_hbm,
             x_vmem, gb_vmem, out_vmem, comm_ref,
             load_sems, store_sems, send_sems, recv_sems):
        my = lax.axis_index("i")

        barrier_sem = pltpu.get_barrier_semaphore()
        for d in range(1, N_DEV):
            pl.semaphore_signal(
                barrier_sem, inc=1,
                device_id=(lax.rem(my + d, N_DEV),),
                device_id_type=pl.DeviceIdType.MESH,
            )

        x_loads = []
        for c in range(N_CHUNK):
            sl = pl.ds(c * rows, rows)
            cp = pltpu.make_async_copy(x_hbm.at[sl], x_vmem.at[sl],
                                       load_sems.at[c])
            cp.start()
            x_loads.append(cp)
        g_load = pltpu.make_async_copy(gamma_hbm, gb_vmem.at[0:1],
                                       load_sems.at[N_CHUNK])
        b_load = pltpu.make_async_copy(beta_hbm, gb_vmem.at[1:2],
                                       load_sems.at[N_CHUNK + 1])
        g_load.start()
        b_load.start()

        psums, psqs = [], []
        for c in range(N_CHUNK):
            x_loads[c].wait()
            xc = x_vmem[c * rows:(c + 1) * rows, :]
            psums.append(jnp.sum(xc, axis=1, keepdims=True))
            psqs.append(jnp.sum(xc * xc, axis=1, keepdims=True))
        stats = jnp.concatenate(
            [jnp.concatenate(psums, axis=0), jnp.concatenate(psqs, axis=0)],
            axis=1)
        comm_ref[0] = stats.T

        pl.semaphore_wait(barrier_sem, N_DEV - 1)

        rdmas = []
        for d in range(1, N_DEV):
            rdma = pltpu.make_async_remote_copy(
                src_ref=comm_ref.at[0],
                dst_ref=comm_ref.at[d],
                send_sem=send_sems.at[d - 1],
                recv_sem=recv_sems.at[d - 1],
                device_id=(lax.rem(my + d, N_DEV),),
                device_id_type=pl.DeviceIdType.MESH,
            )
            rdma.start()
            rdmas.append(rdma)
        g_load.wait()
        b_load.wait()
        for rdma in rdmas:
            rdma.wait_recv()

        total = comm_ref[0] + comm_ref[1] + comm_ref[2] + comm_ref[3]
        total = total.T
        mean = total[:, 0:1] / N_GLOBAL
        var = total[:, 1:2] / N_GLOBAL - mean * mean
        rstd = lax.rsqrt(var + EPS)
        g = gb_vmem[0:1, :]
        b = gb_vmem[1:2, :]

        stores = []
        for c in range(N_CHUNK):
            lo, hi = c * rows, (c + 1) * rows
            xc = x_vmem[lo:hi, :]
            out_vmem[lo:hi, :] = g * ((xc - mean[lo:hi, :]) * rstd[lo:hi, :]) + b
            sl = pl.ds(lo, rows)
            st = pltpu.make_async_copy(out_vmem.at[sl], out_hbm.at[sl],
                                       store_sems.at[c])
            st.start()
            stores.append(st)
        for rdma in rdmas:
            rdma.wait_send()
        for st in stores:
            st.wait()

    return pl.pallas_call(
        body,
        out_shape=jax.ShapeDtypeStruct((m, n_per), x.dtype),
        in_specs=[
            pl.BlockSpec(memory_space=pl.ANY),
            pl.BlockSpec(memory_space=pl.ANY),
            pl.BlockSpec(memory_space=pl.ANY),
        ],
        out_specs=pl.BlockSpec(memory_space=pl.ANY),
        scratch_shapes=[
            pltpu.VMEM((m, n_per), jnp.float32),
            pltpu.VMEM((2, n_per), jnp.float32),
            pltpu.VMEM((m, n_per), jnp.float32),
            pltpu.VMEM((N_DEV, 2, m), jnp.float32),
            pltpu.SemaphoreType.DMA((N_CHUNK + 2,)),
            pltpu.SemaphoreType.DMA((N_CHUNK,)),
            pltpu.SemaphoreType.DMA((N_DEV - 1,)),
            pltpu.SemaphoreType.DMA((N_DEV - 1,)),
        ],
        compiler_params=pltpu.CompilerParams(collective_id=0),
    )(x, gamma.reshape(1, n_per), beta.reshape(1, n_per))
